# baseline (speedup 1.0000x reference)
import functools

import jax
import jax.numpy as jnp
from jax import lax
from jax.experimental import pallas as pl
from jax.experimental.pallas import tpu as pltpu
from jax.experimental.pallas import tpu_sc as plsc

FT_DIM = 256
FT_PAIRS = FT_DIM // 2
PIECE_HEX_DIM = 720
P1_CUTOFF = 360
NSPARSE = 6
CHUNK = 16
RING = 3


def _sc_ft_kernel(B, n_workers):
    per_w = B // n_workers
    n_chunks = per_w // CHUNK
    mesh = plsc.VectorSubcoreMesh(core_axis_name="c", subcore_axis_name="s")

    @functools.partial(
        pl.kernel,
        mesh=mesh,
        compiler_params=pltpu.CompilerParams(needs_layout_passes=False),
        out_type=jax.ShapeDtypeStruct((B, 2 * FT_PAIRS), jnp.int32),
        scratch_types=[
            pltpu.VMEM((PIECE_HEX_DIM, FT_PAIRS), jnp.int32),
            pltpu.VMEM((per_w * NSPARSE // 16, 16), jnp.int32),
            pltpu.VMEM((per_w,), jnp.int32),
            pltpu.VMEM((RING * CHUNK, 2 * FT_PAIRS), jnp.int32),
            pltpu.SemaphoreType.DMA,
        ],
    )
    def k(tab_hbm, sparse_hbm, stm_hbm, out_hbm, tab_v, sp_v, stm_v,
          out_v, osem):
        nc = 2
        wid = lax.axis_index("s") * nc + lax.axis_index("c")
        row_base = wid * per_w
        pltpu.sync_copy(tab_hbm, tab_v)
        pltpu.sync_copy(sparse_hbm.at[wid], sp_v)
        pltpu.sync_copy(stm_hbm.at[wid], stm_v)
        lanes = lax.iota(jnp.int32, 16)
        zero_bf = jnp.zeros((32,), jnp.bfloat16)
        one = jnp.ones((CHUNK,), jnp.int32)
        zero_i = jnp.zeros((CHUNK,), jnp.int32)

        def chunk_body(c, _):
            base = c * CHUNK
            slot = c % RING

            @pl.when(c >= RING)
            def _():
                pltpu.make_async_copy(
                    out_v.at[pl.ds(slot * CHUNK, CHUNK), :],
                    out_hbm.at[pl.ds(0, CHUNK), :], osem).wait()

            stm1 = stm_v[pl.ds(base, CHUNK)]
            flat0 = (lanes + base) * NSPARSE
            cnt_s = zero_i
            cnt_n = zero_i
            slots = [zero_i] * 6
            for j in range(NSPARSE):
                flat = flat0 + j
                s = plsc.load_gather(sp_v, [flat >> 4, flat & 15])
                isp1 = jnp.where(s < P1_CUTOFF, one, zero_i)
                ist = isp1 ^ stm1
                sel_s = (ist == 1) & (cnt_s < 3)
                sel_n = (ist == 0) & (cnt_n < 3)
                for kk in range(3):
                    slots[kk] = jnp.where(sel_s & (cnt_s == kk), s, slots[kk])
                    slots[3 + kk] = jnp.where(sel_n & (cnt_n == kk), s,
                                              slots[3 + kk])
                cnt_s = cnt_s + ist
                cnt_n = cnt_n + (one - ist)

            for b in range(CHUNK):
                srow = slot * CHUNK + b
                ss = [slots[kk][b] for kk in range(6)]
                for l in range(FT_PAIRS // 16):
                    sl = pl.ds(l * 16, 16)
                    g = [plsc.bitcast(tab_v[ss[kk], sl], jnp.bfloat16)
                         for kk in range(6)]
                    acc_s = jnp.maximum((g[0] + g[1]) + g[2], zero_bf)
                    acc_n = jnp.maximum((g[3] + g[4]) + g[5], zero_bf)
                    out_v[srow, sl] = plsc.bitcast(acc_s, jnp.int32)
                    out_v[srow, pl.ds(FT_PAIRS + l * 16, 16)] = plsc.bitcast(
                        acc_n, jnp.int32)

            pltpu.async_copy(
                out_v.at[pl.ds(slot * CHUNK, CHUNK), :],
                out_hbm.at[pl.ds(row_base + base, CHUNK), :], osem)
            return ()

        lax.fori_loop(0, n_chunks, chunk_body, (), unroll=2)
        for _ in range(RING):
            pltpu.make_async_copy(
                out_v.at[pl.ds(0, CHUNK), :],
                out_hbm.at[pl.ds(0, CHUNK), :], osem).wait()

    return k


def _tc_head_kernel(x1_ref, xd_ref, we_ref, wo_ref, w1d_ref, b1_ref, wv_ref,
                    bv_ref, wp_ref, bp_ref, pol_ref, val_ref):
    xi = x1_ref[...]
    fe = jax.lax.bitcast_convert_type(jnp.left_shift(xi, 16), jnp.float32)
    fo = jax.lax.bitcast_convert_type(
        jnp.bitwise_and(xi, jnp.int32(-65536)), jnp.float32)
    h = jnp.dot(fe, we_ref[...], preferred_element_type=jnp.float32)
    h = h + jnp.dot(fo, wo_ref[...], preferred_element_type=jnp.float32)
    h = h + jnp.dot(xd_ref[...], w1d_ref[...],
                    preferred_element_type=jnp.float32)
    h = jnp.maximum(h + b1_ref[...], 0.0)
    pol_ref[...] = jnp.dot(h, wp_ref[...],
                           preferred_element_type=jnp.float32) + bp_ref[...]
    val_ref[...] = jnp.tanh(
        jnp.dot(h, wv_ref[...], preferred_element_type=jnp.float32)
        + bv_ref[...])


def kernel(sparse_batch, dense_batch, stm_players, ft_w, ft_b, W1, b1, Wv, bv,
           Wp, bp):
    B, _ = sparse_batch.shape
    n_workers = 32
    tab_bf = (ft_w + ft_b[None, :] / 3.0).astype(jnp.bfloat16)
    tab_packed = jax.lax.bitcast_convert_type(
        tab_bf.reshape(PIECE_HEX_DIM, FT_PAIRS, 2), jnp.int32)

    sp_r = sparse_batch.reshape(n_workers, B // n_workers * NSPARSE // 16, 16)
    stm_g = stm_players.reshape(n_workers, B // n_workers)
    ft_packed = _sc_ft_kernel(B, n_workers)(tab_packed, sp_r, stm_g)

    blk = 1024
    grid = (B // blk,)
    hid = W1.shape[1]
    ddim = dense_batch.shape[1]
    npol = Wp.shape[1]
    w_even = W1[0:2 * FT_DIM:2]
    w_odd = W1[1:2 * FT_DIM:2]
    pol, val = pl.pallas_call(
        _tc_head_kernel,
        grid=grid,
        in_specs=[
            pl.BlockSpec((blk, FT_DIM), lambda i: (i, 0)),
            pl.BlockSpec((blk, ddim), lambda i: (i, 0)),
            pl.BlockSpec((FT_DIM, hid), lambda i: (0, 0)),
            pl.BlockSpec((FT_DIM, hid), lambda i: (0, 0)),
            pl.BlockSpec((ddim, hid), lambda i: (0, 0)),
            pl.BlockSpec((1, hid), lambda i: (0, 0)),
            pl.BlockSpec((hid, 1), lambda i: (0, 0)),
            pl.BlockSpec((1, 1), lambda i: (0, 0)),
            pl.BlockSpec((hid, npol), lambda i: (0, 0)),
            pl.BlockSpec((1, npol), lambda i: (0, 0)),
        ],
        out_specs=[
            pl.BlockSpec((blk, npol), lambda i: (i, 0)),
            pl.BlockSpec((blk, 1), lambda i: (i, 0)),
        ],
        out_shape=[
            jax.ShapeDtypeStruct((B, npol), jnp.float32),
            jax.ShapeDtypeStruct((B, 1), jnp.float32),
        ],
    )(ft_packed, dense_batch, w_even, w_odd, W1[2 * FT_DIM:], b1[None], Wv,
      bv[None], Wp, bp[None])
    return pol, val[:, 0]


# speedup vs baseline: 1.0837x; 1.0837x over previous
import functools

import jax
import jax.numpy as jnp
from jax import lax
from jax.experimental import pallas as pl
from jax.experimental.pallas import tpu as pltpu
from jax.experimental.pallas import tpu_sc as plsc

FT_DIM = 256
FT_PAIRS = FT_DIM // 2
PIECE_HEX_DIM = 720
P1_CUTOFF = 360
NSPARSE = 6
CHUNK = 16
RING = 3


def _sc_ft_kernel(B, n_workers):
    per_w = B // n_workers
    n_chunks = per_w // CHUNK
    mesh = plsc.VectorSubcoreMesh(core_axis_name="c", subcore_axis_name="s")

    @functools.partial(
        pl.kernel,
        mesh=mesh,
        compiler_params=pltpu.CompilerParams(needs_layout_passes=False),
        out_type=jax.ShapeDtypeStruct((B, 2 * FT_PAIRS), jnp.int32),
        scratch_types=[
            pltpu.VMEM((PIECE_HEX_DIM, FT_PAIRS), jnp.int32),
            pltpu.VMEM((per_w * NSPARSE // 16, 16), jnp.int32),
            pltpu.VMEM((per_w,), jnp.int32),
            pltpu.VMEM((RING * CHUNK, 2 * FT_PAIRS), jnp.int32),
            pltpu.SemaphoreType.DMA,
        ],
    )
    def k(tab_hbm, sparse_hbm, stm_hbm, out_hbm, tab_v, sp_v, stm_v,
          out_v, osem):
        nc = 2
        wid = lax.axis_index("s") * nc + lax.axis_index("c")
        row_base = wid * per_w
        pltpu.sync_copy(tab_hbm, tab_v)
        pltpu.sync_copy(sparse_hbm.at[wid], sp_v)
        pltpu.sync_copy(stm_hbm.at[wid], stm_v)
        lanes = lax.iota(jnp.int32, 16)
        zero_bf = jnp.zeros((32,), jnp.bfloat16)
        one = jnp.ones((CHUNK,), jnp.int32)
        zero_i = jnp.zeros((CHUNK,), jnp.int32)

        def chunk_body(c, _):
            base = c * CHUNK
            slot = c % RING

            @pl.when(c >= RING)
            def _():
                pltpu.make_async_copy(
                    out_v.at[pl.ds(slot * CHUNK, CHUNK), :],
                    out_hbm.at[pl.ds(0, CHUNK), :], osem).wait()

            stm1 = stm_v[pl.ds(base, CHUNK)]
            flat0 = (lanes + base) * NSPARSE
            cnt_s = zero_i
            cnt_n = zero_i
            slots = [zero_i] * 6
            for j in range(NSPARSE):
                flat = flat0 + j
                s = plsc.load_gather(sp_v, [flat >> 4, flat & 15])
                isp1 = jnp.where(s < P1_CUTOFF, one, zero_i)
                ist = isp1 ^ stm1
                sel_s = (ist == 1) & (cnt_s < 3)
                sel_n = (ist == 0) & (cnt_n < 3)
                for kk in range(3):
                    slots[kk] = jnp.where(sel_s & (cnt_s == kk), s, slots[kk])
                    slots[3 + kk] = jnp.where(sel_n & (cnt_n == kk), s,
                                              slots[3 + kk])
                cnt_s = cnt_s + ist
                cnt_n = cnt_n + (one - ist)

            for b in range(CHUNK):
                srow = slot * CHUNK + b
                ss = [slots[kk][b] for kk in range(6)]

                def fblock(l, _):
                    sl = pl.ds(l * 16, 16)
                    g = [plsc.bitcast(tab_v[ss[kk], sl], jnp.bfloat16)
                         for kk in range(6)]
                    acc_s = jnp.maximum((g[0] + g[1]) + g[2], zero_bf)
                    acc_n = jnp.maximum((g[3] + g[4]) + g[5], zero_bf)
                    out_v[srow, sl] = plsc.bitcast(acc_s, jnp.int32)
                    out_v[srow, pl.ds(FT_PAIRS + l * 16, 16)] = plsc.bitcast(
                        acc_n, jnp.int32)
                    return ()

                lax.fori_loop(0, FT_PAIRS // 16, fblock, (), unroll=2)

            pltpu.async_copy(
                out_v.at[pl.ds(slot * CHUNK, CHUNK), :],
                out_hbm.at[pl.ds(row_base + base, CHUNK), :], osem)
            return ()

        lax.fori_loop(0, n_chunks, chunk_body, (), unroll=False)
        for _ in range(RING):
            pltpu.make_async_copy(
                out_v.at[pl.ds(0, CHUNK), :],
                out_hbm.at[pl.ds(0, CHUNK), :], osem).wait()

    return k


def _tc_head_kernel(x1_ref, xd_ref, we_ref, wo_ref, w1d_ref, b1_ref, wv_ref,
                    bv_ref, wp_ref, bp_ref, pol_ref, val_ref):
    xi = x1_ref[...]
    fe = jax.lax.bitcast_convert_type(jnp.left_shift(xi, 16), jnp.float32)
    fo = jax.lax.bitcast_convert_type(
        jnp.bitwise_and(xi, jnp.int32(-65536)), jnp.float32)
    h = jnp.dot(fe, we_ref[...], preferred_element_type=jnp.float32)
    h = h + jnp.dot(fo, wo_ref[...], preferred_element_type=jnp.float32)
    h = h + jnp.dot(xd_ref[...], w1d_ref[...],
                    preferred_element_type=jnp.float32)
    h = jnp.maximum(h + b1_ref[...], 0.0)
    pol_ref[...] = jnp.dot(h, wp_ref[...],
                           preferred_element_type=jnp.float32) + bp_ref[...]
    val_ref[...] = jnp.tanh(
        jnp.dot(h, wv_ref[...], preferred_element_type=jnp.float32)
        + bv_ref[...])


def kernel(sparse_batch, dense_batch, stm_players, ft_w, ft_b, W1, b1, Wv, bv,
           Wp, bp):
    B, _ = sparse_batch.shape
    n_workers = 32
    tab_bf = (ft_w + ft_b[None, :] / 3.0).astype(jnp.bfloat16)
    tab_packed = jax.lax.bitcast_convert_type(
        tab_bf.reshape(PIECE_HEX_DIM, FT_PAIRS, 2), jnp.int32)

    sp_r = sparse_batch.reshape(n_workers, B // n_workers * NSPARSE // 16, 16)
    stm_g = stm_players.reshape(n_workers, B // n_workers)
    ft_packed = _sc_ft_kernel(B, n_workers)(tab_packed, sp_r, stm_g)

    blk = 1024
    grid = (B // blk,)
    hid = W1.shape[1]
    ddim = dense_batch.shape[1]
    npol = Wp.shape[1]
    w_even = W1[0:2 * FT_DIM:2]
    w_odd = W1[1:2 * FT_DIM:2]
    pol, val = pl.pallas_call(
        _tc_head_kernel,
        grid=grid,
        in_specs=[
            pl.BlockSpec((blk, FT_DIM), lambda i: (i, 0)),
            pl.BlockSpec((blk, ddim), lambda i: (i, 0)),
            pl.BlockSpec((FT_DIM, hid), lambda i: (0, 0)),
            pl.BlockSpec((FT_DIM, hid), lambda i: (0, 0)),
            pl.BlockSpec((ddim, hid), lambda i: (0, 0)),
            pl.BlockSpec((1, hid), lambda i: (0, 0)),
            pl.BlockSpec((hid, 1), lambda i: (0, 0)),
            pl.BlockSpec((1, 1), lambda i: (0, 0)),
            pl.BlockSpec((hid, npol), lambda i: (0, 0)),
            pl.BlockSpec((1, npol), lambda i: (0, 0)),
        ],
        out_specs=[
            pl.BlockSpec((blk, npol), lambda i: (i, 0)),
            pl.BlockSpec((blk, 1), lambda i: (i, 0)),
        ],
        out_shape=[
            jax.ShapeDtypeStruct((B, npol), jnp.float32),
            jax.ShapeDtypeStruct((B, 1), jnp.float32),
        ],
    )(ft_packed, dense_batch, w_even, w_odd, W1[2 * FT_DIM:], b1[None], Wv,
      bv[None], Wp, bp[None])
    return pol, val[:, 0]


# speedup vs baseline: 1.1039x; 1.0186x over previous
import functools

import jax
import jax.numpy as jnp
from jax import lax
from jax.experimental import pallas as pl
from jax.experimental.pallas import tpu as pltpu
from jax.experimental.pallas import tpu_sc as plsc

FT_DIM = 256
FT_PAIRS = FT_DIM // 2
PIECE_HEX_DIM = 720
P1_CUTOFF = 360
NSPARSE = 6
CHUNK = 16
RING = 3


def _sc_ft_kernel(B, n_workers):
    per_w = B // n_workers
    n_chunks = per_w // CHUNK
    mesh = plsc.VectorSubcoreMesh(core_axis_name="c", subcore_axis_name="s")

    @functools.partial(
        pl.kernel,
        mesh=mesh,
        compiler_params=pltpu.CompilerParams(needs_layout_passes=False),
        out_type=jax.ShapeDtypeStruct((B, 2 * FT_PAIRS), jnp.int32),
        scratch_types=[
            pltpu.VMEM((PIECE_HEX_DIM, FT_PAIRS), jnp.int32),
            pltpu.VMEM((per_w * NSPARSE // 16, 16), jnp.int32),
            pltpu.VMEM((per_w,), jnp.int32),
            pltpu.VMEM((RING * CHUNK, 2 * FT_PAIRS), jnp.int32),
            pltpu.SemaphoreType.DMA,
        ],
    )
    def k(tab_hbm, sparse_hbm, stm_hbm, out_hbm, tab_v, sp_v, stm_v,
          out_v, osem):
        nc = 2
        wid = lax.axis_index("s") * nc + lax.axis_index("c")
        row_base = wid * per_w
        tab_copy = pltpu.async_copy(tab_hbm, tab_v, osem)
        pltpu.sync_copy(sparse_hbm.at[wid], sp_v)
        pltpu.sync_copy(stm_hbm.at[wid], stm_v)
        tab_copy.wait()
        lanes = lax.iota(jnp.int32, 16)
        zero_bf = jnp.zeros((32,), jnp.bfloat16)
        one = jnp.ones((CHUNK,), jnp.int32)
        zero_i = jnp.zeros((CHUNK,), jnp.int32)

        def chunk_body(c, _):
            base = c * CHUNK
            slot = c % RING

            @pl.when(c >= RING)
            def _():
                pltpu.make_async_copy(
                    out_v.at[pl.ds(slot * CHUNK, CHUNK), :],
                    out_hbm.at[pl.ds(0, CHUNK), :], osem).wait()

            stm1 = stm_v[pl.ds(base, CHUNK)]
            flat0 = (lanes + base) * NSPARSE
            cnt_s = zero_i
            cnt_n = zero_i
            slots = [zero_i] * 6
            for j in range(NSPARSE):
                flat = flat0 + j
                s = plsc.load_gather(sp_v, [flat >> 4, flat & 15])
                isp1 = jnp.where(s < P1_CUTOFF, one, zero_i)
                ist = isp1 ^ stm1
                sel_s = (ist == 1) & (cnt_s < 3)
                sel_n = (ist == 0) & (cnt_n < 3)
                for kk in range(3):
                    slots[kk] = jnp.where(sel_s & (cnt_s == kk), s, slots[kk])
                    slots[3 + kk] = jnp.where(sel_n & (cnt_n == kk), s,
                                              slots[3 + kk])
                cnt_s = cnt_s + ist
                cnt_n = cnt_n + (one - ist)

            for b in range(CHUNK):
                srow = slot * CHUNK + b
                ss = [slots[kk][b] for kk in range(6)]

                def fblock(l, _):
                    sl = pl.ds(l * 16, 16)
                    g = [plsc.bitcast(tab_v[ss[kk], sl], jnp.bfloat16)
                         for kk in range(6)]
                    acc_s = jnp.maximum((g[0] + g[1]) + g[2], zero_bf)
                    acc_n = jnp.maximum((g[3] + g[4]) + g[5], zero_bf)
                    out_v[srow, sl] = plsc.bitcast(acc_s, jnp.int32)
                    out_v[srow, pl.ds(FT_PAIRS + l * 16, 16)] = plsc.bitcast(
                        acc_n, jnp.int32)
                    return ()

                lax.fori_loop(0, FT_PAIRS // 16, fblock, (), unroll=2)

            pltpu.async_copy(
                out_v.at[pl.ds(slot * CHUNK, CHUNK), :],
                out_hbm.at[pl.ds(row_base + base, CHUNK), :], osem)
            return ()

        lax.fori_loop(0, n_chunks, chunk_body, (), unroll=False)
        for _ in range(RING):
            pltpu.make_async_copy(
                out_v.at[pl.ds(0, CHUNK), :],
                out_hbm.at[pl.ds(0, CHUNK), :], osem).wait()

    return k


def _tc_head_kernel(x1_ref, xd_ref, we_ref, wo_ref, w1d_ref, b1_ref, wv_ref,
                    bv_ref, wp_ref, bp_ref, pol_ref, val_ref):
    xi = x1_ref[...]
    fe = jax.lax.bitcast_convert_type(jnp.left_shift(xi, 16), jnp.float32)
    fo = jax.lax.bitcast_convert_type(
        jnp.bitwise_and(xi, jnp.int32(-65536)), jnp.float32)
    h = jnp.dot(fe, we_ref[...], preferred_element_type=jnp.float32)
    h = h + jnp.dot(fo, wo_ref[...], preferred_element_type=jnp.float32)
    h = h + jnp.dot(xd_ref[...], w1d_ref[...],
                    preferred_element_type=jnp.float32)
    h = jnp.maximum(h + b1_ref[...], 0.0)
    pol_ref[...] = jnp.dot(h, wp_ref[...],
                           preferred_element_type=jnp.float32) + bp_ref[...]
    val_ref[...] = jnp.tanh(
        jnp.dot(h, wv_ref[...], preferred_element_type=jnp.float32)
        + bv_ref[...])


def kernel(sparse_batch, dense_batch, stm_players, ft_w, ft_b, W1, b1, Wv, bv,
           Wp, bp):
    B, _ = sparse_batch.shape
    n_workers = 32
    tab_bf = (ft_w + ft_b[None, :] / 3.0).astype(jnp.bfloat16)
    tab_packed = jax.lax.bitcast_convert_type(
        tab_bf.reshape(PIECE_HEX_DIM, FT_PAIRS, 2), jnp.int32)

    sp_r = sparse_batch.reshape(n_workers, B // n_workers * NSPARSE // 16, 16)
    stm_g = stm_players.reshape(n_workers, B // n_workers)
    ft_packed = _sc_ft_kernel(B, n_workers)(tab_packed, sp_r, stm_g)

    blk = 1024
    grid = (B // blk,)
    hid = W1.shape[1]
    ddim = dense_batch.shape[1]
    npol = Wp.shape[1]
    w_even = W1[0:2 * FT_DIM:2]
    w_odd = W1[1:2 * FT_DIM:2]
    pol, val = pl.pallas_call(
        _tc_head_kernel,
        grid=grid,
        in_specs=[
            pl.BlockSpec((blk, FT_DIM), lambda i: (i, 0)),
            pl.BlockSpec((blk, ddim), lambda i: (i, 0)),
            pl.BlockSpec((FT_DIM, hid), lambda i: (0, 0)),
            pl.BlockSpec((FT_DIM, hid), lambda i: (0, 0)),
            pl.BlockSpec((ddim, hid), lambda i: (0, 0)),
            pl.BlockSpec((1, hid), lambda i: (0, 0)),
            pl.BlockSpec((hid, 1), lambda i: (0, 0)),
            pl.BlockSpec((1, 1), lambda i: (0, 0)),
            pl.BlockSpec((hid, npol), lambda i: (0, 0)),
            pl.BlockSpec((1, npol), lambda i: (0, 0)),
        ],
        out_specs=[
            pl.BlockSpec((blk, npol), lambda i: (i, 0)),
            pl.BlockSpec((blk, 1), lambda i: (i, 0)),
        ],
        out_shape=[
            jax.ShapeDtypeStruct((B, npol), jnp.float32),
            jax.ShapeDtypeStruct((B, 1), jnp.float32),
        ],
    )(ft_packed, dense_batch, w_even, w_odd, W1[2 * FT_DIM:], b1[None], Wv,
      bv[None], Wp, bp[None])
    return pol, val[:, 0]
